# baseline (speedup 1.0000x reference)
import functools

import jax
import jax.numpy as jnp
from jax import lax
from jax.experimental import pallas as pl
from jax.experimental.pallas import tpu as pltpu
from jax.experimental.pallas import tpu_sc as plsc

NC = 2
NS = 16
NW = NC * NS
CH = 80
G = 64

_MESH = dict(core_axis_name="c", subcore_axis_name="s")


NB = 3


def _zero_acc_slice(zbuf, acc, base_r, rpt, h):
    nz = zbuf.shape[0]
    for r in range(nz):
        for c in range(h // 16):
            zbuf[r, pl.ds(c * 16, 16)] = jnp.zeros((16,), jnp.float32)
    for k in range(rpt // nz):
        pltpu.sync_copy(zbuf, acc.at[pl.ds(base_r + k * nz, nz)])


def _write_out_slice(acc, out_hbm, n, base_r, rpt, sid, cid):
    tail = n - (NS - 1) * rpt

    @pl.when(sid < NS - 1)
    def _():
        pltpu.sync_copy(acc.at[pl.ds(base_r, rpt)],
                        out_hbm.at[pl.ds(cid * n + base_r, rpt)])

    @pl.when(sid == NS - 1)
    def _():
        pltpu.sync_copy(acc.at[pl.ds(base_r, tail)],
                        out_hbm.at[pl.ds(cid * n + base_r, tail)])


def _deg_body(n, npad, ncht, dst2_hbm, ones_hbm, out_hbm,
              idst, ones_v, zrow_v, acc, isem, ssem):
    cid = lax.axis_index("c")
    sid = lax.axis_index("s")
    w = cid * NS + sid
    rpt = npad // NS
    nch = ncht
    cp_i = pltpu.async_copy(dst2_hbm.at[pl.ds(w * ncht, ncht)], idst, isem)
    cp_o = pltpu.async_copy(ones_hbm, ones_v, isem)
    _zero_acc_slice(zrow_v, acc, sid * rpt, rpt, 128)
    cp_i.wait()
    cp_o.wait()
    plsc.subcore_barrier()

    def body(j, carry):
        pltpu.async_copy(ones_v, acc.at[idst.at[j, 0]], ssem, add=True)
        return carry

    lax.fori_loop(0, nch, body, 0)

    def drain(j, carry):
        pltpu.make_async_copy(ones_v, acc.at[idst.at[0, 0]], ssem).wait()
        return carry

    lax.fori_loop(0, nch, drain, 0)
    plsc.subcore_barrier()
    _write_out_slice(acc, out_hbm, n, sid * rpt, rpt, sid, cid)


def _agg_body(n, npad, h, epws, g_hbm, src_hbm, dst2_hbm, out_hbm,
              isrc, idst, rows, acc, *sems):
    epw0, epw1 = epws
    gsem = list(sems[:NB])
    ssem = list(sems[NB:2 * NB])
    isem = list(sems[2 * NB:])
    cid = lax.axis_index("c")
    sid = lax.axis_index("s")
    rpt = npad // NS
    ebase = jnp.where(cid == 0, sid * epw0, NS * epw0 + sid * epw1)
    epw = jnp.where(cid == 0, epw0, epw1)
    nch = epw // CH
    cbase = ebase // CH

    @pl.when(cid == 0)
    def _():
        pltpu.async_copy(src_hbm.at[pl.ds(ebase, epw0)], isrc.at[pl.ds(0, epw0)],
                         gsem[0])

    @pl.when(cid == 1)
    def _():
        pltpu.async_copy(src_hbm.at[pl.ds(ebase, epw1)], isrc.at[pl.ds(0, epw1)],
                         gsem[0])

    _zero_acc_slice(rows.at[0], acc, sid * rpt, rpt, h)

    @pl.when(cid == 0)
    def _():
        pltpu.make_async_copy(src_hbm.at[pl.ds(0, epw0)],
                              isrc.at[pl.ds(0, epw0)], gsem[0]).wait()

    @pl.when(cid == 1)
    def _():
        pltpu.make_async_copy(src_hbm.at[pl.ds(0, epw1)],
                              isrc.at[pl.ds(0, epw1)], gsem[0]).wait()

    plsc.subcore_barrier()

    for b in range(NB):
        pltpu.async_copy(dst2_hbm.at[cbase + b], idst.at[b], isem[b])
        pltpu.async_copy(g_hbm.at[isrc.at[pl.ds(b * CH, CH)]],
                         rows.at[b], gsem[b])

    def outer(t, carry):
        for b in range(NB):
            j = t * NB + b
            pltpu.make_async_copy(dst2_hbm.at[cbase], idst.at[b],
                                  isem[b]).wait()
            pltpu.make_async_copy(g_hbm.at[isrc.at[pl.ds(0, CH)]],
                                  rows.at[b], gsem[b]).wait()
            pltpu.async_copy(rows.at[b], acc.at[idst.at[b, 0]], ssem[b],
                             add=True)
            pltpu.make_async_copy(rows.at[b], acc.at[idst.at[0, 0]],
                                  ssem[b]).wait()

            @pl.when(t < nch // NB - 1)
            def _():
                pltpu.async_copy(dst2_hbm.at[cbase + j + NB], idst.at[b],
                                 isem[b])
                pltpu.async_copy(
                    g_hbm.at[isrc.at[pl.ds((j + NB) * CH, CH)]],
                    rows.at[b], gsem[b])
        return carry

    lax.fori_loop(0, nch // NB, outer, 0)
    plsc.subcore_barrier()
    _write_out_slice(acc, out_hbm, n, sid * rpt, rpt, sid, cid)


F0_NUM, F0_DEN = 1, 3


def _split_epw(e):
    q = NB * CH
    epw0 = (e * F0_NUM // (F0_DEN * NS)) // q * q
    epw1 = -(-(e - NS * epw0) // (NS * q)) * q
    return epw0, epw1


def _pad_edges(src, dst, n, npad):
    e = src.shape[0]
    epw0, epw1 = _split_epw(e)
    pad = NS * (epw0 + epw1) - e
    src_p = jnp.concatenate([src, jnp.zeros((pad,), jnp.int32)])
    dst_p = jnp.concatenate(
        [dst, n + (jnp.arange(pad, dtype=jnp.int32) % (npad - n))])
    nchunks = NS * (epw0 + epw1) // CH
    return src_p, dst_p.reshape(nchunks, 1, CH), (epw0, epw1)


def _run_deg_sc(dst2, n, npad, epws):
    ncht = dst2.shape[0] // NW
    ones = jnp.ones((CH, 128), jnp.float32)
    body = functools.partial(_deg_body, n, npad, ncht)
    f = pl.kernel(
        body,
        out_type=jax.ShapeDtypeStruct((2 * n, 128), jnp.float32),
        mesh=plsc.VectorSubcoreMesh(**_MESH),
        scratch_types=[
            pltpu.VMEM((ncht, 1, CH), jnp.int32),
            pltpu.VMEM((CH, 128), jnp.float32),
            pltpu.VMEM((128, 128), jnp.float32),
            pltpu.VMEM_SHARED((npad, 128), jnp.float32),
            pltpu.SemaphoreType.DMA,
            pltpu.SemaphoreType.DMA,
        ],
    )
    return f(dst2, ones)


def _run_agg_sc(g, src_p, dst2, n, npad, epws):
    h = g.shape[1]
    body = functools.partial(_agg_body, n, npad, h, epws)
    f = pl.kernel(
        body,
        out_type=jax.ShapeDtypeStruct((2 * n, h), jnp.float32),
        mesh=plsc.VectorSubcoreMesh(**_MESH),
        scratch_types=[
            pltpu.VMEM((max(epws),), jnp.int32),
            pltpu.VMEM((NB, 1, CH), jnp.int32),
            pltpu.VMEM((NB, CH, h), jnp.float32),
            pltpu.VMEM_SHARED((npad, h), jnp.float32),
        ] + [pltpu.SemaphoreType.DMA] * (3 * NB),
    )
    return f(g, src_p, dst2)


BLK = 1000


def _first_body(d0, d1, x, w, g_out, dinv_out):
    deg = d0[:, 0:1] + d1[:, 0:1] + 1.0
    dinv = lax.rsqrt(deg)
    g_out[...] = dinv * jnp.dot(x[...], w[...],
                                preferred_element_type=jnp.float32)
    dinv_out[...] = dinv


def _mid_body(g, s0, s1, dinv_r, b, w, g_out):
    dinv = dinv_r[...]
    hrelu = jnp.maximum(dinv * (g[...] + s0[...] + s1[...]) + b[...], 0.0)
    g_out[...] = dinv * jnp.dot(hrelu, w[...],
                                preferred_element_type=jnp.float32)


def _final_body(g, s0, s1, dinv_r, b, ids_r, wl, bl, out, sums, cnt):
    i = pl.program_id(0)

    @pl.when(i == 0)
    def _():
        sums[...] = jnp.zeros_like(sums)
        cnt[...] = jnp.zeros_like(cnt)

    dinv = dinv_r[...]
    hh = jnp.maximum(dinv * (g[...] + s0[...] + s1[...]) + b[...], 0.0)
    iota = lax.broadcasted_iota(jnp.int32, (BLK, G), 1)
    onehot = (ids_r[...] == iota).astype(jnp.float32)
    dn = (((0,), (0,)), ((), ()))
    sums[...] += lax.dot_general(onehot, hh, dn,
                                 preferred_element_type=jnp.float32)
    cnt[...] += lax.dot_general(onehot, jnp.ones_like(hh), dn,
                                preferred_element_type=jnp.float32)

    @pl.when(i == pl.num_programs(0) - 1)
    def _():
        pooled = sums[...] / jnp.maximum(cnt[...], 1.0)
        out[...] = jnp.dot(pooled, wl[...],
                           preferred_element_type=jnp.float32) + bl[...]


def _run_first_tc(deg2, x, w1):
    n, f_in = x.shape
    hdim = w1.shape[1]
    nb = n // BLK
    return pl.pallas_call(
        _first_body,
        grid=(nb,),
        in_specs=[
            pl.BlockSpec((BLK, 128), lambda i: (i, 0)),
            pl.BlockSpec((BLK, 128), lambda i, _nb=nb: (i + _nb, 0)),
            pl.BlockSpec((BLK, f_in), lambda i: (i, 0)),
            pl.BlockSpec((f_in, hdim), lambda i: (0, 0)),
        ],
        out_specs=[
            pl.BlockSpec((BLK, hdim), lambda i: (i, 0)),
            pl.BlockSpec((BLK, 1), lambda i: (i, 0)),
        ],
        out_shape=[
            jax.ShapeDtypeStruct((n, hdim), jnp.float32),
            jax.ShapeDtypeStruct((n, 1), jnp.float32),
        ],
    )(deg2, deg2, x, w1)


def _run_mid_tc(g, s2, dinv, b, w):
    n, hdim = g.shape
    nb = n // BLK
    return pl.pallas_call(
        _mid_body,
        grid=(nb,),
        in_specs=[
            pl.BlockSpec((BLK, hdim), lambda i: (i, 0)),
            pl.BlockSpec((BLK, hdim), lambda i: (i, 0)),
            pl.BlockSpec((BLK, hdim), lambda i, _nb=nb: (i + _nb, 0)),
            pl.BlockSpec((BLK, 1), lambda i: (i, 0)),
            pl.BlockSpec((1, hdim), lambda i: (0, 0)),
            pl.BlockSpec((hdim, hdim), lambda i: (0, 0)),
        ],
        out_specs=pl.BlockSpec((BLK, hdim), lambda i: (i, 0)),
        out_shape=jax.ShapeDtypeStruct((n, hdim), jnp.float32),
    )(g, s2, s2, dinv, b.reshape(1, hdim), w)


def _run_final_tc(g, s2, dinv, b, ids2, wl, bl):
    n, hdim = g.shape
    t = wl.shape[1]
    nb = n // BLK
    return pl.pallas_call(
        _final_body,
        grid=(nb,),
        in_specs=[
            pl.BlockSpec((BLK, hdim), lambda i: (i, 0)),
            pl.BlockSpec((BLK, hdim), lambda i: (i, 0)),
            pl.BlockSpec((BLK, hdim), lambda i, _nb=nb: (i + _nb, 0)),
            pl.BlockSpec((BLK, 1), lambda i: (i, 0)),
            pl.BlockSpec((1, hdim), lambda i: (0, 0)),
            pl.BlockSpec((BLK, 1), lambda i: (i, 0)),
            pl.BlockSpec((hdim, t), lambda i: (0, 0)),
            pl.BlockSpec((1, t), lambda i: (0, 0)),
        ],
        out_specs=pl.BlockSpec((G, t), lambda i: (0, 0)),
        out_shape=jax.ShapeDtypeStruct((G, t), jnp.float32),
        scratch_shapes=[
            pltpu.VMEM((G, hdim), jnp.float32),
            pltpu.VMEM((G, hdim), jnp.float32),
        ],
    )(g, s2, s2, dinv, b.reshape(1, hdim), ids2, wl, bl.reshape(1, t))


def kernel(x, edge_index, batch, W1, b1, W2, b2, W3, b3, Wl, bl):

    x = x.astype(jnp.float32)
    n = x.shape[0]
    npad = -(-n // (NS * 128)) * (NS * 128)
    src_p, dst3, epw = _pad_edges(edge_index[0], edge_index[1], n, npad)

    deg2 = _run_deg_sc(dst3, n, npad, epw)
    g1, dinv = _run_first_tc(deg2, x, W1)
    s1 = _run_agg_sc(g1, src_p, dst3, n, npad, epw)
    g2 = _run_mid_tc(g1, s1, dinv, b1, W2)
    s2 = _run_agg_sc(g2, src_p, dst3, n, npad, epw)
    g3 = _run_mid_tc(g2, s2, dinv, b2, W3)
    s3 = _run_agg_sc(g3, src_p, dst3, n, npad, epw)
    ids2 = batch.reshape(n, 1)
    return _run_final_tc(g3, s3, dinv, b3, ids2, Wl, bl)


# speedup vs baseline: 1.1396x; 1.1396x over previous
import functools

import jax
import jax.numpy as jnp
from jax import lax
from jax.experimental import pallas as pl
from jax.experimental.pallas import tpu as pltpu
from jax.experimental.pallas import tpu_sc as plsc

NC = 2
NS = 16
NW = NC * NS
CH = 80
G = 64

_MESH = dict(core_axis_name="c", subcore_axis_name="s")


NB = 3


def _zero_acc_slice(zbuf, acc, base_r, rpt, h):
    nz = zbuf.shape[0]
    for r in range(nz):
        for c in range(h // 16):
            zbuf[r, pl.ds(c * 16, 16)] = jnp.zeros((16,), jnp.float32)
    for k in range(rpt // nz):
        pltpu.sync_copy(zbuf, acc.at[pl.ds(base_r + k * nz, nz)])


def _write_out_slice(acc, out_hbm, n, base_r, rpt, sid, cid):
    tail = n - (NS - 1) * rpt

    @pl.when(sid < NS - 1)
    def _():
        pltpu.sync_copy(acc.at[pl.ds(base_r, rpt)],
                        out_hbm.at[pl.ds(cid * n + base_r, rpt)])

    @pl.when(sid == NS - 1)
    def _():
        pltpu.sync_copy(acc.at[pl.ds(base_r, tail)],
                        out_hbm.at[pl.ds(cid * n + base_r, tail)])


def _deg_body(n, npad, ncht, dst2_hbm, ones_hbm, out_hbm,
              idst, ones_v, zrow_v, acc, isem, ssem):
    cid = lax.axis_index("c")
    sid = lax.axis_index("s")
    w = cid * NS + sid
    rpt = npad // NS
    nch = ncht
    cp_i = pltpu.async_copy(dst2_hbm.at[pl.ds(w * ncht, ncht)], idst, isem)
    cp_o = pltpu.async_copy(ones_hbm, ones_v, isem)
    _zero_acc_slice(zrow_v, acc, sid * rpt, rpt, 128)
    cp_i.wait()
    cp_o.wait()
    plsc.subcore_barrier()

    def body(j, carry):
        pltpu.async_copy(ones_v, acc.at[idst.at[j, 0]], ssem, add=True)
        return carry

    lax.fori_loop(0, nch, body, 0)

    def drain(j, carry):
        pltpu.make_async_copy(ones_v, acc.at[idst.at[0, 0]], ssem).wait()
        return carry

    lax.fori_loop(0, nch, drain, 0)
    plsc.subcore_barrier()
    _write_out_slice(acc, out_hbm, n, sid * rpt, rpt, sid, cid)


def _agg_body(n, npad, h, epws, g_hbm, src_hbm, dst2_hbm, out_hbm,
              isrc, idst, rows, acc, *sems):
    epw0, epw1 = epws
    gsem = list(sems[:NB])
    ssem = list(sems[NB:2 * NB])
    isem = list(sems[2 * NB:])
    cid = lax.axis_index("c")
    sid = lax.axis_index("s")
    rpt = npad // NS
    ebase = jnp.where(cid == 0, sid * epw0, NS * epw0 + sid * epw1)
    epw = jnp.where(cid == 0, epw0, epw1)
    nch = epw // CH
    cbase = ebase // CH

    @pl.when(cid == 0)
    def _():
        pltpu.async_copy(src_hbm.at[pl.ds(ebase, epw0)], isrc.at[pl.ds(0, epw0)],
                         gsem[0])

    @pl.when(cid == 1)
    def _():
        pltpu.async_copy(src_hbm.at[pl.ds(ebase, epw1)], isrc.at[pl.ds(0, epw1)],
                         gsem[0])

    _zero_acc_slice(rows.at[0], acc, sid * rpt, rpt, h)

    @pl.when(cid == 0)
    def _():
        pltpu.make_async_copy(src_hbm.at[pl.ds(0, epw0)],
                              isrc.at[pl.ds(0, epw0)], gsem[0]).wait()

    @pl.when(cid == 1)
    def _():
        pltpu.make_async_copy(src_hbm.at[pl.ds(0, epw1)],
                              isrc.at[pl.ds(0, epw1)], gsem[0]).wait()

    plsc.subcore_barrier()

    for b in range(NB):
        pltpu.async_copy(dst2_hbm.at[cbase + b], idst.at[b], isem[b])
        pltpu.async_copy(g_hbm.at[isrc.at[pl.ds(b * CH, CH)]],
                         rows.at[b], gsem[b])

    def outer(t, carry):
        for b in range(NB):
            j = t * NB + b
            pltpu.make_async_copy(dst2_hbm.at[cbase], idst.at[b],
                                  isem[b]).wait()
            pltpu.make_async_copy(g_hbm.at[isrc.at[pl.ds(0, CH)]],
                                  rows.at[b], gsem[b]).wait()
            pltpu.async_copy(rows.at[b], acc.at[idst.at[b, 0]], ssem[b],
                             add=True)
            pltpu.make_async_copy(rows.at[b], acc.at[idst.at[0, 0]],
                                  ssem[b]).wait()

            @pl.when(t < nch // NB - 1)
            def _():
                pltpu.async_copy(dst2_hbm.at[cbase + j + NB], idst.at[b],
                                 isem[b])
                pltpu.async_copy(
                    g_hbm.at[isrc.at[pl.ds((j + NB) * CH, CH)]],
                    rows.at[b], gsem[b])
        return carry

    lax.fori_loop(0, nch // NB, outer, 0)
    plsc.subcore_barrier()
    _write_out_slice(acc, out_hbm, n, sid * rpt, rpt, sid, cid)


F0_NUM, F0_DEN = 2, 3


def _split_epw(e):
    q = NB * CH
    epw0 = (e * F0_NUM // (F0_DEN * NS)) // q * q
    epw1 = -(-(e - NS * epw0) // (NS * q)) * q
    return epw0, epw1


def _pad_edges(src, dst, n, npad):
    e = src.shape[0]
    epw0, epw1 = _split_epw(e)
    pad = NS * (epw0 + epw1) - e
    src_p = jnp.concatenate([src, jnp.zeros((pad,), jnp.int32)])
    dst_p = jnp.concatenate(
        [dst, n + (jnp.arange(pad, dtype=jnp.int32) % (npad - n))])
    nchunks = NS * (epw0 + epw1) // CH
    return src_p, dst_p.reshape(nchunks, 1, CH), (epw0, epw1)


def _run_deg_sc(dst2, n, npad, epws):
    ncht = dst2.shape[0] // NW
    ones = jnp.ones((CH, 128), jnp.float32)
    body = functools.partial(_deg_body, n, npad, ncht)
    f = pl.kernel(
        body,
        out_type=jax.ShapeDtypeStruct((2 * n, 128), jnp.float32),
        mesh=plsc.VectorSubcoreMesh(**_MESH),
        scratch_types=[
            pltpu.VMEM((ncht, 1, CH), jnp.int32),
            pltpu.VMEM((CH, 128), jnp.float32),
            pltpu.VMEM((128, 128), jnp.float32),
            pltpu.VMEM_SHARED((npad, 128), jnp.float32),
            pltpu.SemaphoreType.DMA,
            pltpu.SemaphoreType.DMA,
        ],
    )
    return f(dst2, ones)


def _run_agg_sc(g, src_p, dst2, n, npad, epws):
    h = g.shape[1]
    body = functools.partial(_agg_body, n, npad, h, epws)
    f = pl.kernel(
        body,
        out_type=jax.ShapeDtypeStruct((2 * n, h), jnp.float32),
        mesh=plsc.VectorSubcoreMesh(**_MESH),
        scratch_types=[
            pltpu.VMEM((max(epws),), jnp.int32),
            pltpu.VMEM((NB, 1, CH), jnp.int32),
            pltpu.VMEM((NB, CH, h), jnp.float32),
            pltpu.VMEM_SHARED((npad, h), jnp.float32),
        ] + [pltpu.SemaphoreType.DMA] * (3 * NB),
    )
    return f(g, src_p, dst2)


BLK = 1000


def _first_body(d0, d1, x, w, g_out, dinv_out):
    deg = d0[:, 0:1] + d1[:, 0:1] + 1.0
    dinv = lax.rsqrt(deg)
    g_out[...] = dinv * jnp.dot(x[...], w[...],
                                preferred_element_type=jnp.float32)
    dinv_out[...] = dinv


def _mid_body(g, s0, s1, dinv_r, b, w, g_out):
    dinv = dinv_r[...]
    hrelu = jnp.maximum(dinv * (g[...] + s0[...] + s1[...]) + b[...], 0.0)
    g_out[...] = dinv * jnp.dot(hrelu, w[...],
                                preferred_element_type=jnp.float32)


def _final_body(g, s0, s1, dinv_r, b, ids_r, wl, bl, out, sums, cnt):
    i = pl.program_id(0)

    @pl.when(i == 0)
    def _():
        sums[...] = jnp.zeros_like(sums)
        cnt[...] = jnp.zeros_like(cnt)

    dinv = dinv_r[...]
    hh = jnp.maximum(dinv * (g[...] + s0[...] + s1[...]) + b[...], 0.0)
    iota = lax.broadcasted_iota(jnp.int32, (BLK, G), 1)
    onehot = (ids_r[...] == iota).astype(jnp.float32)
    dn = (((0,), (0,)), ((), ()))
    sums[...] += lax.dot_general(onehot, hh, dn,
                                 preferred_element_type=jnp.float32)
    cnt[...] += lax.dot_general(onehot, jnp.ones_like(hh), dn,
                                preferred_element_type=jnp.float32)

    @pl.when(i == pl.num_programs(0) - 1)
    def _():
        pooled = sums[...] / jnp.maximum(cnt[...], 1.0)
        out[...] = jnp.dot(pooled, wl[...],
                           preferred_element_type=jnp.float32) + bl[...]


def _run_first_tc(deg2, x, w1):
    n, f_in = x.shape
    hdim = w1.shape[1]
    nb = n // BLK
    return pl.pallas_call(
        _first_body,
        grid=(nb,),
        in_specs=[
            pl.BlockSpec((BLK, 128), lambda i: (i, 0)),
            pl.BlockSpec((BLK, 128), lambda i, _nb=nb: (i + _nb, 0)),
            pl.BlockSpec((BLK, f_in), lambda i: (i, 0)),
            pl.BlockSpec((f_in, hdim), lambda i: (0, 0)),
        ],
        out_specs=[
            pl.BlockSpec((BLK, hdim), lambda i: (i, 0)),
            pl.BlockSpec((BLK, 1), lambda i: (i, 0)),
        ],
        out_shape=[
            jax.ShapeDtypeStruct((n, hdim), jnp.float32),
            jax.ShapeDtypeStruct((n, 1), jnp.float32),
        ],
    )(deg2, deg2, x, w1)


def _run_mid_tc(g, s2, dinv, b, w):
    n, hdim = g.shape
    nb = n // BLK
    return pl.pallas_call(
        _mid_body,
        grid=(nb,),
        in_specs=[
            pl.BlockSpec((BLK, hdim), lambda i: (i, 0)),
            pl.BlockSpec((BLK, hdim), lambda i: (i, 0)),
            pl.BlockSpec((BLK, hdim), lambda i, _nb=nb: (i + _nb, 0)),
            pl.BlockSpec((BLK, 1), lambda i: (i, 0)),
            pl.BlockSpec((1, hdim), lambda i: (0, 0)),
            pl.BlockSpec((hdim, hdim), lambda i: (0, 0)),
        ],
        out_specs=pl.BlockSpec((BLK, hdim), lambda i: (i, 0)),
        out_shape=jax.ShapeDtypeStruct((n, hdim), jnp.float32),
    )(g, s2, s2, dinv, b.reshape(1, hdim), w)


def _run_final_tc(g, s2, dinv, b, ids2, wl, bl):
    n, hdim = g.shape
    t = wl.shape[1]
    nb = n // BLK
    return pl.pallas_call(
        _final_body,
        grid=(nb,),
        in_specs=[
            pl.BlockSpec((BLK, hdim), lambda i: (i, 0)),
            pl.BlockSpec((BLK, hdim), lambda i: (i, 0)),
            pl.BlockSpec((BLK, hdim), lambda i, _nb=nb: (i + _nb, 0)),
            pl.BlockSpec((BLK, 1), lambda i: (i, 0)),
            pl.BlockSpec((1, hdim), lambda i: (0, 0)),
            pl.BlockSpec((BLK, 1), lambda i: (i, 0)),
            pl.BlockSpec((hdim, t), lambda i: (0, 0)),
            pl.BlockSpec((1, t), lambda i: (0, 0)),
        ],
        out_specs=pl.BlockSpec((G, t), lambda i: (0, 0)),
        out_shape=jax.ShapeDtypeStruct((G, t), jnp.float32),
        scratch_shapes=[
            pltpu.VMEM((G, hdim), jnp.float32),
            pltpu.VMEM((G, hdim), jnp.float32),
        ],
    )(g, s2, s2, dinv, b.reshape(1, hdim), ids2, wl, bl.reshape(1, t))


def kernel(x, edge_index, batch, W1, b1, W2, b2, W3, b3, Wl, bl):

    x = x.astype(jnp.float32)
    n = x.shape[0]
    npad = -(-n // (NS * 128)) * (NS * 128)
    src_p, dst3, epw = _pad_edges(edge_index[0], edge_index[1], n, npad)

    deg2 = _run_deg_sc(dst3, n, npad, epw)
    g1, dinv = _run_first_tc(deg2, x, W1)
    s1 = _run_agg_sc(g1, src_p, dst3, n, npad, epw)
    g2 = _run_mid_tc(g1, s1, dinv, b1, W2)
    s2 = _run_agg_sc(g2, src_p, dst3, n, npad, epw)
    g3 = _run_mid_tc(g2, s2, dinv, b2, W3)
    s3 = _run_agg_sc(g3, src_p, dst3, n, npad, epw)
    ids2 = batch.reshape(n, 1)
    return _run_final_tc(g3, s3, dinv, b3, ids2, Wl, bl)


# speedup vs baseline: 1.1452x; 1.0049x over previous
import functools

import jax
import jax.numpy as jnp
from jax import lax
from jax.experimental import pallas as pl
from jax.experimental.pallas import tpu as pltpu
from jax.experimental.pallas import tpu_sc as plsc

NC = 2
NS = 16
NW = NC * NS
CH = 80
G = 64

_MESH = dict(core_axis_name="c", subcore_axis_name="s")


NB = 3


def _zero_acc_slice(zbuf, acc, base_r, rpt, h):
    nz = zbuf.shape[0]
    for r in range(nz):
        for c in range(h // 16):
            zbuf[r, pl.ds(c * 16, 16)] = jnp.zeros((16,), jnp.float32)
    for k in range(rpt // nz):
        pltpu.sync_copy(zbuf, acc.at[pl.ds(base_r + k * nz, nz)])


def _write_out_slice(acc, out_hbm, n, base_r, rpt, sid, cid):
    tail = n - (NS - 1) * rpt

    @pl.when(sid < NS - 1)
    def _():
        pltpu.sync_copy(acc.at[pl.ds(base_r, rpt)],
                        out_hbm.at[pl.ds(cid * n + base_r, rpt)])

    @pl.when(sid == NS - 1)
    def _():
        pltpu.sync_copy(acc.at[pl.ds(base_r, tail)],
                        out_hbm.at[pl.ds(cid * n + base_r, tail)])


def _deg_body(n, npad, ncht, dst2_hbm, ones_hbm, out_hbm,
              idst, ones_v, zrow_v, acc, isem, ssem):
    cid = lax.axis_index("c")
    sid = lax.axis_index("s")
    w = cid * NS + sid
    rpt = npad // NS
    nch = ncht
    cp_i = pltpu.async_copy(dst2_hbm.at[pl.ds(w * ncht, ncht)], idst, isem)
    cp_o = pltpu.async_copy(ones_hbm, ones_v, isem)
    _zero_acc_slice(zrow_v, acc, sid * rpt, rpt, 128)
    cp_i.wait()
    cp_o.wait()
    plsc.subcore_barrier()

    def body(j, carry):
        pltpu.async_copy(ones_v, acc.at[idst.at[j, 0]], ssem, add=True)
        return carry

    lax.fori_loop(0, nch, body, 0)

    def drain(j, carry):
        pltpu.make_async_copy(ones_v, acc.at[idst.at[0, 0]], ssem).wait()
        return carry

    lax.fori_loop(0, nch, drain, 0)
    plsc.subcore_barrier()
    _write_out_slice(acc, out_hbm, n, sid * rpt, rpt, sid, cid)


def _agg_body(n, npad, h, epws, g_hbm, src_hbm, dst2_hbm, out_hbm,
              isrc, idst, rows, acc, *sems):
    epw0, epw1 = epws
    gsem = list(sems[:NB])
    ssem = list(sems[NB:2 * NB])
    isem = list(sems[2 * NB:])
    cid = lax.axis_index("c")
    sid = lax.axis_index("s")
    rpt = npad // NS
    ebase = jnp.where(cid == 0, sid * epw0, NS * epw0 + sid * epw1)
    epw = jnp.where(cid == 0, epw0, epw1)
    nch = epw // CH
    cbase = ebase // CH

    @pl.when(cid == 0)
    def _():
        pltpu.async_copy(src_hbm.at[pl.ds(ebase, epw0)], isrc.at[pl.ds(0, epw0)],
                         gsem[0])

    @pl.when(cid == 1)
    def _():
        pltpu.async_copy(src_hbm.at[pl.ds(ebase, epw1)], isrc.at[pl.ds(0, epw1)],
                         gsem[0])

    _zero_acc_slice(rows.at[0], acc, sid * rpt, rpt, h)

    @pl.when(cid == 0)
    def _():
        pltpu.make_async_copy(src_hbm.at[pl.ds(0, epw0)],
                              isrc.at[pl.ds(0, epw0)], gsem[0]).wait()

    @pl.when(cid == 1)
    def _():
        pltpu.make_async_copy(src_hbm.at[pl.ds(0, epw1)],
                              isrc.at[pl.ds(0, epw1)], gsem[0]).wait()

    plsc.subcore_barrier()

    for b in range(NB):
        pltpu.async_copy(dst2_hbm.at[cbase + b], idst.at[b], isem[b])
        pltpu.async_copy(g_hbm.at[isrc.at[pl.ds(b * CH, CH)]],
                         rows.at[b], gsem[b])

    def outer(t, carry):
        for b in range(NB):
            j = t * NB + b
            pltpu.make_async_copy(dst2_hbm.at[cbase], idst.at[b],
                                  isem[b]).wait()
            pltpu.make_async_copy(g_hbm.at[isrc.at[pl.ds(0, CH)]],
                                  rows.at[b], gsem[b]).wait()
            pltpu.async_copy(rows.at[b], acc.at[idst.at[b, 0]], ssem[b],
                             add=True)
            pltpu.make_async_copy(rows.at[b], acc.at[idst.at[0, 0]],
                                  ssem[b]).wait()

            @pl.when(t < nch // NB - 1)
            def _():
                pltpu.async_copy(dst2_hbm.at[cbase + j + NB], idst.at[b],
                                 isem[b])
                pltpu.async_copy(
                    g_hbm.at[isrc.at[pl.ds((j + NB) * CH, CH)]],
                    rows.at[b], gsem[b])
        return carry

    lax.fori_loop(0, nch // NB, outer, 0)
    plsc.subcore_barrier()
    _write_out_slice(acc, out_hbm, n, sid * rpt, rpt, sid, cid)


F0_NUM, F0_DEN = 27, 40


def _split_epw(e):
    q = NB * CH
    epw0 = (e * F0_NUM // (F0_DEN * NS)) // q * q
    epw1 = -(-(e - NS * epw0) // (NS * q)) * q
    return epw0, epw1


def _pad_edges(src, dst, n, npad):
    e = src.shape[0]
    epw0, epw1 = _split_epw(e)
    pad = NS * (epw0 + epw1) - e
    src_p = jnp.concatenate([src, jnp.zeros((pad,), jnp.int32)])
    dst_p = jnp.concatenate(
        [dst, n + (jnp.arange(pad, dtype=jnp.int32) % (npad - n))])
    nchunks = NS * (epw0 + epw1) // CH
    return src_p, dst_p.reshape(nchunks, 1, CH), (epw0, epw1)


def _run_deg_sc(dst2, n, npad, epws):
    ncht = dst2.shape[0] // NW
    ones = jnp.ones((CH, 128), jnp.float32)
    body = functools.partial(_deg_body, n, npad, ncht)
    f = pl.kernel(
        body,
        out_type=jax.ShapeDtypeStruct((2 * n, 128), jnp.float32),
        mesh=plsc.VectorSubcoreMesh(**_MESH),
        scratch_types=[
            pltpu.VMEM((ncht, 1, CH), jnp.int32),
            pltpu.VMEM((CH, 128), jnp.float32),
            pltpu.VMEM((128, 128), jnp.float32),
            pltpu.VMEM_SHARED((npad, 128), jnp.float32),
            pltpu.SemaphoreType.DMA,
            pltpu.SemaphoreType.DMA,
        ],
    )
    return f(dst2, ones)


def _run_agg_sc(g, src_p, dst2, n, npad, epws):
    h = g.shape[1]
    body = functools.partial(_agg_body, n, npad, h, epws)
    f = pl.kernel(
        body,
        out_type=jax.ShapeDtypeStruct((2 * n, h), jnp.float32),
        mesh=plsc.VectorSubcoreMesh(**_MESH),
        scratch_types=[
            pltpu.VMEM((max(epws),), jnp.int32),
            pltpu.VMEM((NB, 1, CH), jnp.int32),
            pltpu.VMEM((NB, CH, h), jnp.float32),
            pltpu.VMEM_SHARED((npad, h), jnp.float32),
        ] + [pltpu.SemaphoreType.DMA] * (3 * NB),
    )
    return f(g, src_p, dst2)


BLK = 1000


def _first_body(d0, d1, x, w, g_out, dinv_out):
    deg = d0[:, 0:1] + d1[:, 0:1] + 1.0
    dinv = lax.rsqrt(deg)
    g_out[...] = dinv * jnp.dot(x[...], w[...],
                                preferred_element_type=jnp.float32)
    dinv_out[...] = dinv


def _mid_body(g, s0, s1, dinv_r, b, w, g_out):
    dinv = dinv_r[...]
    hrelu = jnp.maximum(dinv * (g[...] + s0[...] + s1[...]) + b[...], 0.0)
    g_out[...] = dinv * jnp.dot(hrelu, w[...],
                                preferred_element_type=jnp.float32)


def _final_body(g, s0, s1, dinv_r, b, ids_r, wl, bl, out, sums, cnt):
    i = pl.program_id(0)

    @pl.when(i == 0)
    def _():
        sums[...] = jnp.zeros_like(sums)
        cnt[...] = jnp.zeros_like(cnt)

    dinv = dinv_r[...]
    hh = jnp.maximum(dinv * (g[...] + s0[...] + s1[...]) + b[...], 0.0)
    iota = lax.broadcasted_iota(jnp.int32, (BLK, G), 1)
    onehot = (ids_r[...] == iota).astype(jnp.float32)
    dn = (((0,), (0,)), ((), ()))
    sums[...] += lax.dot_general(onehot, hh, dn,
                                 preferred_element_type=jnp.float32)
    cnt[...] += lax.dot_general(onehot, jnp.ones_like(hh), dn,
                                preferred_element_type=jnp.float32)

    @pl.when(i == pl.num_programs(0) - 1)
    def _():
        pooled = sums[...] / jnp.maximum(cnt[...], 1.0)
        out[...] = jnp.dot(pooled, wl[...],
                           preferred_element_type=jnp.float32) + bl[...]


def _run_first_tc(deg2, x, w1):
    n, f_in = x.shape
    hdim = w1.shape[1]
    nb = n // BLK
    return pl.pallas_call(
        _first_body,
        grid=(nb,),
        in_specs=[
            pl.BlockSpec((BLK, 128), lambda i: (i, 0)),
            pl.BlockSpec((BLK, 128), lambda i, _nb=nb: (i + _nb, 0)),
            pl.BlockSpec((BLK, f_in), lambda i: (i, 0)),
            pl.BlockSpec((f_in, hdim), lambda i: (0, 0)),
        ],
        out_specs=[
            pl.BlockSpec((BLK, hdim), lambda i: (i, 0)),
            pl.BlockSpec((BLK, 1), lambda i: (i, 0)),
        ],
        out_shape=[
            jax.ShapeDtypeStruct((n, hdim), jnp.float32),
            jax.ShapeDtypeStruct((n, 1), jnp.float32),
        ],
    )(deg2, deg2, x, w1)


def _run_mid_tc(g, s2, dinv, b, w):
    n, hdim = g.shape
    nb = n // BLK
    return pl.pallas_call(
        _mid_body,
        grid=(nb,),
        in_specs=[
            pl.BlockSpec((BLK, hdim), lambda i: (i, 0)),
            pl.BlockSpec((BLK, hdim), lambda i: (i, 0)),
            pl.BlockSpec((BLK, hdim), lambda i, _nb=nb: (i + _nb, 0)),
            pl.BlockSpec((BLK, 1), lambda i: (i, 0)),
            pl.BlockSpec((1, hdim), lambda i: (0, 0)),
            pl.BlockSpec((hdim, hdim), lambda i: (0, 0)),
        ],
        out_specs=pl.BlockSpec((BLK, hdim), lambda i: (i, 0)),
        out_shape=jax.ShapeDtypeStruct((n, hdim), jnp.float32),
    )(g, s2, s2, dinv, b.reshape(1, hdim), w)


def _run_final_tc(g, s2, dinv, b, ids2, wl, bl):
    n, hdim = g.shape
    t = wl.shape[1]
    nb = n // BLK
    return pl.pallas_call(
        _final_body,
        grid=(nb,),
        in_specs=[
            pl.BlockSpec((BLK, hdim), lambda i: (i, 0)),
            pl.BlockSpec((BLK, hdim), lambda i: (i, 0)),
            pl.BlockSpec((BLK, hdim), lambda i, _nb=nb: (i + _nb, 0)),
            pl.BlockSpec((BLK, 1), lambda i: (i, 0)),
            pl.BlockSpec((1, hdim), lambda i: (0, 0)),
            pl.BlockSpec((BLK, 1), lambda i: (i, 0)),
            pl.BlockSpec((hdim, t), lambda i: (0, 0)),
            pl.BlockSpec((1, t), lambda i: (0, 0)),
        ],
        out_specs=pl.BlockSpec((G, t), lambda i: (0, 0)),
        out_shape=jax.ShapeDtypeStruct((G, t), jnp.float32),
        scratch_shapes=[
            pltpu.VMEM((G, hdim), jnp.float32),
            pltpu.VMEM((G, hdim), jnp.float32),
        ],
    )(g, s2, s2, dinv, b.reshape(1, hdim), ids2, wl, bl.reshape(1, t))


def kernel(x, edge_index, batch, W1, b1, W2, b2, W3, b3, Wl, bl):

    x = x.astype(jnp.float32)
    n = x.shape[0]
    npad = -(-n // (NS * 128)) * (NS * 128)
    src_p, dst3, epw = _pad_edges(edge_index[0], edge_index[1], n, npad)

    deg2 = _run_deg_sc(dst3, n, npad, epw)
    g1, dinv = _run_first_tc(deg2, x, W1)
    s1 = _run_agg_sc(g1, src_p, dst3, n, npad, epw)
    g2 = _run_mid_tc(g1, s1, dinv, b1, W2)
    s2 = _run_agg_sc(g2, src_p, dst3, n, npad, epw)
    g3 = _run_mid_tc(g2, s2, dinv, b2, W3)
    s3 = _run_agg_sc(g3, src_p, dst3, n, npad, epw)
    ids2 = batch.reshape(n, 1)
    return _run_final_tc(g3, s3, dinv, b3, ids2, Wl, bl)
